# baseline (speedup 1.0000x reference)
import jax
import jax.numpy as jnp
from jax.experimental import pallas as pl

MAX_DEPTH = 20.0
VALID_THRESHOLD = 1e-06
MIN_VALID_POINTS = 10


def _body(x_ref, y_ref, o_ref):
    x = x_ref[0]
    y = y_ref[0]
    h = x.shape[0]
    mask = (y > VALID_THRESHOLD) & (y <= MAX_DEPTH)
    mf = jnp.where(mask, 1.0, 0.0)
    xm = jnp.where(mask, x, 0.0)
    ym = jnp.where(mask, y, 0.0)
    xxm = xm * xm
    xym = xm * ym
    ones = jnp.full((8, h), 1.0, dtype=x.dtype)
    parts = [
        jax.lax.dot_general(
            ones, s, (((1,), (0,)), ((), ())),
            preferred_element_type=jnp.float32,
        )
        for s in (mf, xm, xxm, ym, xym)
    ]
    sums = jnp.stack([jnp.sum(p) for p in parts]) / 8.0
    n = sums[0]
    x_sum = sums[1]
    x_sq_sum = sums[2]
    y_sum = sums[3]
    xy_sum = sums[4]
    det = n * x_sq_sum - x_sum * x_sum
    valid = (n >= MIN_VALID_POINTS) & (jnp.abs(det) >= 1e-08)
    safe_det = jnp.where(valid, det, 1.0)
    scale = jnp.where(valid, (n * xy_sum - x_sum * y_sum) / safe_det, 1.0)
    bias = jnp.where(valid, (x_sq_sum * y_sum - x_sum * xy_sum) / safe_det, 0.0)
    o_ref[0] = scale * x + bias


def kernel(non_scale_dense, sparse_depth):
    b, c, h, w = non_scale_dense.shape
    x = non_scale_dense.reshape(b, h, w)
    y = sparse_depth.reshape(b, h, w)
    out = pl.pallas_call(
        _body,
        grid=(b,),
        in_specs=[
            pl.BlockSpec((1, h, w), lambda i: (i, 0, 0)),
            pl.BlockSpec((1, h, w), lambda i: (i, 0, 0)),
        ],
        out_specs=pl.BlockSpec((1, h, w), lambda i: (i, 0, 0)),
        out_shape=jax.ShapeDtypeStruct((b, h, w), x.dtype),
    )(x, y)
    return out.reshape(b, c, h, w)


# speedup vs baseline: 1.2850x; 1.2850x over previous
import jax
import jax.numpy as jnp
from jax.experimental import pallas as pl

MAX_DEPTH = 20.0
VALID_THRESHOLD = 1e-06
MIN_VALID_POINTS = 10
BLOCK_B = 4


def _body(x_ref, y_ref, o_ref):
    h = x_ref.shape[1]
    ones = jnp.full((8, h), 1.0, dtype=x_ref.dtype)
    for bi in range(BLOCK_B):
        x = x_ref[bi]
        y = y_ref[bi]
        mask = (y > VALID_THRESHOLD) & (y <= MAX_DEPTH)
        mf = jnp.where(mask, 1.0, 0.0)
        xm = jnp.where(mask, x, 0.0)
        ym = jnp.where(mask, y, 0.0)
        xxm = xm * xm
        xym = xm * ym
        parts = [
            jax.lax.dot_general(
                ones, s, (((1,), (0,)), ((), ())),
                preferred_element_type=jnp.float32,
            )
            for s in (mf, xm, xxm, ym, xym)
        ]
        sums = jnp.stack([jnp.sum(p) for p in parts]) / 8.0
        n = sums[0]
        x_sum = sums[1]
        x_sq_sum = sums[2]
        y_sum = sums[3]
        xy_sum = sums[4]
        det = n * x_sq_sum - x_sum * x_sum
        valid = (n >= MIN_VALID_POINTS) & (jnp.abs(det) >= 1e-08)
        safe_det = jnp.where(valid, det, 1.0)
        scale = jnp.where(valid, (n * xy_sum - x_sum * y_sum) / safe_det, 1.0)
        bias = jnp.where(valid, (x_sq_sum * y_sum - x_sum * xy_sum) / safe_det, 0.0)
        o_ref[bi] = scale * x + bias


def kernel(non_scale_dense, sparse_depth):
    b, c, h, w = non_scale_dense.shape
    x = non_scale_dense.reshape(b, h, w)
    y = sparse_depth.reshape(b, h, w)
    out = pl.pallas_call(
        _body,
        grid=(b // BLOCK_B,),
        in_specs=[
            pl.BlockSpec((BLOCK_B, h, w), lambda i: (i, 0, 0)),
            pl.BlockSpec((BLOCK_B, h, w), lambda i: (i, 0, 0)),
        ],
        out_specs=pl.BlockSpec((BLOCK_B, h, w), lambda i: (i, 0, 0)),
        out_shape=jax.ShapeDtypeStruct((b, h, w), x.dtype),
    )(x, y)
    return out.reshape(b, c, h, w)


# speedup vs baseline: 1.2869x; 1.0015x over previous
import jax
import jax.numpy as jnp
from jax.experimental import pallas as pl

MAX_DEPTH = 20.0
VALID_THRESHOLD = 1e-06
MIN_VALID_POINTS = 10
BLOCK_B = 4


def _body(x_ref, y_ref, o_ref):
    h = x_ref.shape[1]
    ones = jnp.full((8, h), 1.0, dtype=jnp.bfloat16)
    scale_bias = []
    for bi in range(BLOCK_B):
        x = x_ref[bi]
        y = y_ref[bi]
        mask = (y > VALID_THRESHOLD) & (y <= MAX_DEPTH)
        mf = jnp.where(mask, 1.0, 0.0)
        xm = jnp.where(mask, x, 0.0)
        ym = jnp.where(mask, y, 0.0)
        xxm = xm * xm
        xym = xm * ym
        parts = [
            jax.lax.dot_general(
                ones, s.astype(jnp.bfloat16), (((1,), (0,)), ((), ())),
                preferred_element_type=jnp.float32,
            )
            for s in (mf, xm, xxm, ym, xym)
        ]
        sums = jnp.sum(jnp.stack(parts), axis=(1, 2)) / 8.0
        n = sums[0]
        x_sum = sums[1]
        x_sq_sum = sums[2]
        y_sum = sums[3]
        xy_sum = sums[4]
        det = n * x_sq_sum - x_sum * x_sum
        valid = (n >= MIN_VALID_POINTS) & (jnp.abs(det) >= 1e-08)
        safe_det = jnp.where(valid, det, 1.0)
        scale = jnp.where(valid, (n * xy_sum - x_sum * y_sum) / safe_det, 1.0)
        bias = jnp.where(valid, (x_sq_sum * y_sum - x_sum * xy_sum) / safe_det, 0.0)
        scale_bias.append((scale, bias))
    for bi in range(BLOCK_B):
        scale, bias = scale_bias[bi]
        o_ref[bi] = scale * x_ref[bi] + bias


def kernel(non_scale_dense, sparse_depth):
    b, c, h, w = non_scale_dense.shape
    x = non_scale_dense.reshape(b, h, w)
    y = sparse_depth.reshape(b, h, w)
    out = pl.pallas_call(
        _body,
        grid=(b // BLOCK_B,),
        in_specs=[
            pl.BlockSpec((BLOCK_B, h, w), lambda i: (i, 0, 0)),
            pl.BlockSpec((BLOCK_B, h, w), lambda i: (i, 0, 0)),
        ],
        out_specs=pl.BlockSpec((BLOCK_B, h, w), lambda i: (i, 0, 0)),
        out_shape=jax.ShapeDtypeStruct((b, h, w), x.dtype),
    )(x, y)
    return out.reshape(b, c, h, w)
